# Initial kernel scaffold; baseline (speedup 1.0000x reference)
import jax
import jax.numpy as jnp
from jax.experimental import pallas as pl


def kernel(x, lut):
    raise NotImplementedError("write your pallas kernel here")


# speedup vs baseline: 1.3478x; 1.3478x over previous
import functools
import math

import jax
import jax.numpy as jnp
import numpy as np
from jax import lax
from jax.experimental import pallas as pl
from jax.experimental.pallas import tpu as pltpu
from jax.experimental.pallas import tpu_sc as plsc

D_MODEL = 768
BATCH = 4
SEQ = 8192
N_TOK = BATCH * SEQ
NUM_WORKERS = 32
B_PER_W = N_TOK // NUM_WORKERS
CHUNK = 64
N_CHUNKS = B_PER_W // CHUNK
LANES = 16
SCALE = math.sqrt(float(D_MODEL))


def _pe_table() -> np.ndarray:
    pos = np.arange(SEQ, dtype=np.float32)[:, None]
    div = np.exp(
        np.arange(0, D_MODEL, 2, dtype=np.float32)
        * (-(math.log(10000.0) / D_MODEL))
    )
    angle = (pos * div).astype(np.float32)
    pe = np.empty((SEQ, D_MODEL), dtype=np.float32)
    pe[:, 0::2] = np.sin(angle)
    pe[:, 1::2] = np.cos(angle)
    return pe


_PE = _pe_table()


def _sc_embed(x_hbm, pe_hbm, lut_hbm, out_hbm, idx_v, rows_v, pe_v, sem):
    wid = lax.axis_index("s") * 2 + lax.axis_index("c")
    base = wid * B_PER_W
    pltpu.sync_copy(x_hbm.at[wid], idx_v)
    pe_base = lax.rem(base, SEQ)

    def chunk_body(c, _):
        pltpu.async_copy(lut_hbm.at[idx_v.at[c]], rows_v, sem).wait()
        pltpu.sync_copy(pe_hbm.at[pl.ds(pe_base + c * CHUNK, CHUNK)], pe_v)

        def row_body(i, _):
            for k in range(D_MODEL // LANES):
                sl = pl.ds(k * LANES, LANES)
                rows_v[i, sl] = rows_v[i, sl] * SCALE + pe_v[i, sl]
            return 0

        lax.fori_loop(0, CHUNK, row_body, 0)
        pltpu.sync_copy(rows_v, out_hbm.at[pl.ds(base + c * CHUNK, CHUNK)])
        return 0

    lax.fori_loop(0, N_CHUNKS, chunk_body, 0)


def kernel(x, lut):
    x_w = x.reshape(NUM_WORKERS, N_CHUNKS, CHUNK).astype(jnp.int32)
    pe = jnp.asarray(_PE)
    run = pl.kernel(
        _sc_embed,
        out_type=jax.ShapeDtypeStruct((N_TOK, D_MODEL), jnp.float32),
        mesh=plsc.VectorSubcoreMesh(core_axis_name="c", subcore_axis_name="s"),
        scratch_types=[
            pltpu.VMEM((N_CHUNKS, CHUNK), jnp.int32),
            pltpu.VMEM((CHUNK, D_MODEL), jnp.float32),
            pltpu.VMEM((CHUNK, D_MODEL), jnp.float32),
            pltpu.SemaphoreType.DMA,
        ],
    )
    out = run(x_w, pe, lut)
    return out.reshape(BATCH, SEQ, D_MODEL)


# speedup vs baseline: 1.9668x; 1.4592x over previous
import math

import jax
import jax.numpy as jnp
import numpy as np
from jax import lax
from jax.experimental import pallas as pl
from jax.experimental.pallas import tpu as pltpu
from jax.experimental.pallas import tpu_sc as plsc

D_MODEL = 768
BATCH = 4
SEQ = 8192
N_TOK = BATCH * SEQ
NUM_WORKERS = 32
B_PER_W = N_TOK // NUM_WORKERS
CHUNK = 16
N_CHUNKS = B_PER_W // CHUNK
LANES = 16
SCALE = math.sqrt(float(D_MODEL))


def _pe_table() -> np.ndarray:
    pos = np.arange(SEQ, dtype=np.float32)[:, None]
    div = np.exp(
        np.arange(0, D_MODEL, 2, dtype=np.float32)
        * (-(math.log(10000.0) / D_MODEL))
    )
    angle = (pos * div).astype(np.float32)
    pe = np.empty((SEQ, D_MODEL), dtype=np.float32)
    pe[:, 0::2] = np.sin(angle)
    pe[:, 1::2] = np.cos(angle)
    return pe


_PE = _pe_table()


def _sc_embed(x_hbm, pe_hbm, lut_hbm, out_hbm,
              idx_v, rows0, rows1, pe0, pe1, out0, out1,
              gsem0, gsem1, psem0, psem1, wsem0, wsem1):
    rows = (rows0, rows1)
    pes = (pe0, pe1)
    outs = (out0, out1)
    gsems = (gsem0, gsem1)
    psems = (psem0, psem1)
    wsems = (wsem0, wsem1)

    wid = lax.axis_index("s") * 2 + lax.axis_index("c")
    base = wid * B_PER_W
    pltpu.sync_copy(x_hbm.at[wid], idx_v)
    pe_base = lax.rem(base, SEQ)

    def start_gather(c, b):
        pltpu.async_copy(lut_hbm.at[idx_v.at[c]], rows[b], gsems[b])
        pltpu.async_copy(
            pe_hbm.at[pl.ds(pe_base + c * CHUNK, CHUNK)], pes[b], psems[b])

    def wait_gather(c, b):
        pltpu.make_async_copy(lut_hbm.at[idx_v.at[c]], rows[b], gsems[b]).wait()
        pltpu.make_async_copy(
            pe_hbm.at[pl.ds(pe_base + c * CHUNK, CHUNK)], pes[b],
            psems[b]).wait()

    def out_copy(c, b):
        return pltpu.make_async_copy(
            outs[b], out_hbm.at[pl.ds(base + c * CHUNK, CHUNK)], wsems[b])

    start_gather(0, 0)
    start_gather(1, 1)

    def step(c, b):
        wait_gather(c, b)

        @pl.when(c >= 2)
        def _():
            out_copy(c - 2, b).wait()

        def row_body(r, _):
            for k in range(D_MODEL // LANES):
                sl = pl.ds(k * LANES, LANES)
                outs[b][r, sl] = rows[b][r, sl] * SCALE + pes[b][r, sl]
            return 0

        lax.fori_loop(0, CHUNK, row_body, 0)
        out_copy(c, b).start()

        @pl.when(c + 2 < N_CHUNKS)
        def _():
            start_gather(c + 2, b)

    def pair(i, _):
        step(i * 2, 0)
        step(i * 2 + 1, 1)
        return 0

    lax.fori_loop(0, N_CHUNKS // 2, pair, 0)
    out_copy(N_CHUNKS - 2, 0).wait()
    out_copy(N_CHUNKS - 1, 1).wait()


def kernel(x, lut):
    x_w = x.reshape(NUM_WORKERS, N_CHUNKS, CHUNK).astype(jnp.int32)
    pe = jnp.asarray(_PE)
    run = pl.kernel(
        _sc_embed,
        out_type=jax.ShapeDtypeStruct((N_TOK, D_MODEL), jnp.float32),
        mesh=plsc.VectorSubcoreMesh(core_axis_name="c", subcore_axis_name="s"),
        scratch_types=[
            pltpu.VMEM((N_CHUNKS, CHUNK), jnp.int32),
            pltpu.VMEM((CHUNK, D_MODEL), jnp.float32),
            pltpu.VMEM((CHUNK, D_MODEL), jnp.float32),
            pltpu.VMEM((CHUNK, D_MODEL), jnp.float32),
            pltpu.VMEM((CHUNK, D_MODEL), jnp.float32),
            pltpu.VMEM((CHUNK, D_MODEL), jnp.float32),
            pltpu.VMEM((CHUNK, D_MODEL), jnp.float32),
            pltpu.SemaphoreType.DMA,
            pltpu.SemaphoreType.DMA,
            pltpu.SemaphoreType.DMA,
            pltpu.SemaphoreType.DMA,
            pltpu.SemaphoreType.DMA,
            pltpu.SemaphoreType.DMA,
        ],
    )
    out = run(x_w, pe, lut)
    return out.reshape(BATCH, SEQ, D_MODEL)
